# initial kernel scaffold (unmeasured)
import functools

import jax
import jax.numpy as jnp
from jax import lax
from jax.experimental import pallas as pl
from jax.experimental.pallas import tpu as pltpu

NZ = 4
CHUNK = 64


def _gather_dest(dest):
    r = dest.shape[0]
    d2 = dest.reshape(32, r // 32)

    def body(d_ref, out_ref, send_sem, recv_sem):
        my_x = lax.axis_index("x")
        my_y = lax.axis_index("y")
        my_z = lax.axis_index("z")

        barrier_sem = pltpu.get_barrier_semaphore()
        for dz in range(NZ):
            @pl.when(dz != my_z)
            def _():
                pl.semaphore_signal(
                    barrier_sem, inc=1,
                    device_id=(my_x, my_y, dz),
                    device_id_type=pl.DeviceIdType.MESH,
                )
        pl.semaphore_wait(barrier_sem, NZ - 1)

        out_ref[my_z, :, :] = d_ref[:, :]
        for dz in range(NZ):
            @pl.when(dz != my_z)
            def _():
                rdma = pltpu.make_async_remote_copy(
                    src_ref=d_ref,
                    dst_ref=out_ref.at[my_z],
                    send_sem=send_sem,
                    recv_sem=recv_sem,
                    device_id=(my_x, my_y, dz),
                    device_id_type=pl.DeviceIdType.MESH,
                )
                rdma.start()

        dummy = pltpu.make_async_remote_copy(
            src_ref=d_ref,
            dst_ref=out_ref.at[0],
            send_sem=send_sem,
            recv_sem=recv_sem,
            device_id=(my_x, my_y, my_z),
            device_id_type=pl.DeviceIdType.MESH,
        )
        for _ in range(NZ - 1):
            dummy.wait_send()
        for _ in range(NZ - 1):
            dummy.wait_recv()

    out = pl.pallas_call(
        body,
        out_shape=jax.ShapeDtypeStruct((NZ, 32, r // 32), jnp.int32),
        in_specs=[pl.BlockSpec(memory_space=pltpu.VMEM)],
        out_specs=pl.BlockSpec(memory_space=pltpu.VMEM),
        scratch_shapes=[pltpu.SemaphoreType.DMA, pltpu.SemaphoreType.DMA],
        compiler_params=pltpu.CompilerParams(collective_id=0),
    )(d2)
    return out.reshape(NZ, r)


def _a2av(x_send, meta):
    rows, d_model = x_send.shape

    def body(xs_ref, meta_ref, out_ref, sb_sem, ss_sem, rb_sem, rs_sem):
        my_x = lax.axis_index("x")
        my_y = lax.axis_index("y")
        my_z = lax.axis_index("z")

        barrier_sem = pltpu.get_barrier_semaphore()
        for dz in range(NZ):
            @pl.when(dz != my_z)
            def _():
                pl.semaphore_signal(
                    barrier_sem, inc=1,
                    device_id=(my_x, my_y, dz),
                    device_id_type=pl.DeviceIdType.MESH,
                )
        pl.semaphore_wait(barrier_sem, NZ - 1)

        for d in range(NZ):
            cnt = meta_ref[d]
            s0 = meta_ref[4 + d]
            t0 = meta_ref[8 + d]
            nfull = cnt // CHUNK
            rem = cnt - nfull * CHUNK

            @pl.when(d == my_z)
            def _():
                def cp_chunk(k, c):
                    out_ref[pl.ds(t0 + k * CHUNK, CHUNK), :] = (
                        xs_ref[pl.ds(s0 + k * CHUNK, CHUNK), :]
                    )
                    return c
                lax.fori_loop(0, nfull, cp_chunk, 0)

                def cp_row(k, c):
                    out_ref[pl.ds(t0 + nfull * CHUNK + k, 1), :] = (
                        xs_ref[pl.ds(s0 + nfull * CHUNK + k, 1), :]
                    )
                    return c
                lax.fori_loop(0, rem, cp_row, 0)

            @pl.when(d != my_z)
            def _():
                def send_chunk(k, c):
                    rdma = pltpu.make_async_remote_copy(
                        src_ref=xs_ref.at[pl.ds(s0 + k * CHUNK, CHUNK)],
                        dst_ref=out_ref.at[pl.ds(t0 + k * CHUNK, CHUNK)],
                        send_sem=sb_sem,
                        recv_sem=rb_sem,
                        device_id=(my_x, my_y, d),
                        device_id_type=pl.DeviceIdType.MESH,
                    )
                    rdma.start()
                    return c
                lax.fori_loop(0, nfull, send_chunk, 0)

                def send_row(k, c):
                    rdma = pltpu.make_async_remote_copy(
                        src_ref=xs_ref.at[pl.ds(s0 + nfull * CHUNK + k, 1)],
                        dst_ref=out_ref.at[pl.ds(t0 + nfull * CHUNK + k, 1)],
                        send_sem=ss_sem,
                        recv_sem=rs_sem,
                        device_id=(my_x, my_y, d),
                        device_id_type=pl.DeviceIdType.MESH,
                    )
                    rdma.start()
                    return c
                lax.fori_loop(0, rem, send_row, 0)

        dummy_big = pltpu.make_async_remote_copy(
            src_ref=xs_ref.at[pl.ds(0, CHUNK)],
            dst_ref=out_ref.at[pl.ds(0, CHUNK)],
            send_sem=sb_sem,
            recv_sem=rb_sem,
            device_id=(my_x, my_y, my_z),
            device_id_type=pl.DeviceIdType.MESH,
        )
        dummy_small = pltpu.make_async_remote_copy(
            src_ref=xs_ref.at[pl.ds(0, 1)],
            dst_ref=out_ref.at[pl.ds(0, 1)],
            send_sem=ss_sem,
            recv_sem=rs_sem,
            device_id=(my_x, my_y, my_z),
            device_id_type=pl.DeviceIdType.MESH,
        )

        def wait_sb(k, c):
            dummy_big.wait_send()
            return c
        lax.fori_loop(0, meta_ref[12], wait_sb, 0)

        def wait_ss(k, c):
            dummy_small.wait_send()
            return c
        lax.fori_loop(0, meta_ref[13], wait_ss, 0)

        def wait_rb(k, c):
            dummy_big.wait_recv()
            return c
        lax.fori_loop(0, meta_ref[14], wait_rb, 0)

        def wait_rs(k, c):
            dummy_small.wait_recv()
            return c
        lax.fori_loop(0, meta_ref[15], wait_rs, 0)

        @functools.partial(
            pl.run_scoped, second_barrier=pltpu.SemaphoreType.REGULAR
        )
        def _(second_barrier):
            for dz in range(NZ):
                @pl.when(dz != my_z)
                def _():
                    pl.semaphore_signal(
                        second_barrier, inc=1,
                        device_id=(my_x, my_y, dz),
                        device_id_type=pl.DeviceIdType.MESH,
                    )
            pl.semaphore_wait(second_barrier, NZ - 1)

    return pl.pallas_call(
        body,
        out_shape=jax.ShapeDtypeStruct((rows, d_model), x_send.dtype),
        in_specs=[
            pl.BlockSpec(memory_space=pltpu.VMEM),
            pl.BlockSpec(memory_space=pltpu.SMEM),
        ],
        out_specs=pl.BlockSpec(memory_space=pltpu.VMEM),
        scratch_shapes=[
            pltpu.SemaphoreType.DMA,
            pltpu.SemaphoreType.DMA,
            pltpu.SemaphoreType.DMA,
            pltpu.SemaphoreType.DMA,
        ],
        compiler_params=pltpu.CompilerParams(collective_id=1),
    )(x_send, meta)


def kernel(x, dest):
    rows = x.shape[0]
    my_z = lax.axis_index("z")

    dest_full = _gather_dest(dest)

    c = (
        dest_full[:, :, None] == jnp.arange(NZ, dtype=dest_full.dtype)[None, None, :]
    ).sum(axis=1).astype(jnp.int32)

    cnt = c[my_z]
    csum = jnp.cumsum(cnt)
    src0 = jnp.concatenate([jnp.zeros((1,), jnp.int32), csum[:-1].astype(jnp.int32)])
    excl = (jnp.cumsum(c, axis=0) - c).astype(jnp.int32)
    dst0 = excl[my_z]

    recv_cnt = c[:, my_z]
    others = jnp.arange(NZ) != my_z
    zero = jnp.zeros((), jnp.int32)
    n_send_big = jnp.where(others, cnt // CHUNK, zero).sum()
    n_send_small = jnp.where(others, cnt % CHUNK, zero).sum()
    n_recv_big = jnp.where(others, recv_cnt // CHUNK, zero).sum()
    n_recv_small = jnp.where(others, recv_cnt % CHUNK, zero).sum()

    meta = jnp.concatenate(
        [
            cnt.astype(jnp.int32),
            src0,
            dst0,
            jnp.stack([n_send_big, n_send_small, n_recv_big, n_recv_small]),
        ]
    ).astype(jnp.int32)

    send_perm = jnp.argsort(dest, stable=True)
    x_send = jnp.take(x, send_perm, axis=0)

    return _a2av(x_send, meta)


# baseline (device time: 281446 ns/iter reference)
import functools

import jax
import jax.numpy as jnp
from jax import lax
from jax.experimental import pallas as pl
from jax.experimental.pallas import tpu as pltpu

NZ = 4
CHUNK = 64


def _gather_dest(dest):
    r = dest.shape[0]
    d2 = dest.reshape(32, r // 32)

    def body(d_ref, out_ref, send_sem, recv_sem):
        my_x = lax.axis_index("x")
        my_y = lax.axis_index("y")
        my_z = lax.axis_index("z")

        barrier_sem = pltpu.get_barrier_semaphore()
        for dz in range(NZ):
            @pl.when(dz != my_z)
            def _():
                pl.semaphore_signal(
                    barrier_sem, inc=1,
                    device_id=(my_x, my_y, dz),
                    device_id_type=pl.DeviceIdType.MESH,
                )
        pl.semaphore_wait(barrier_sem, NZ - 1)

        out_ref[my_z, :, :] = d_ref[:, :]
        for dz in range(NZ):
            @pl.when(dz != my_z)
            def _():
                rdma = pltpu.make_async_remote_copy(
                    src_ref=d_ref,
                    dst_ref=out_ref.at[my_z],
                    send_sem=send_sem,
                    recv_sem=recv_sem,
                    device_id=(my_x, my_y, dz),
                    device_id_type=pl.DeviceIdType.MESH,
                )
                rdma.start()

        dummy = pltpu.make_async_remote_copy(
            src_ref=d_ref,
            dst_ref=out_ref.at[0],
            send_sem=send_sem,
            recv_sem=recv_sem,
            device_id=(my_x, my_y, my_z),
            device_id_type=pl.DeviceIdType.MESH,
        )
        for _ in range(NZ - 1):
            dummy.wait_send()
        for _ in range(NZ - 1):
            dummy.wait_recv()

    out = pl.pallas_call(
        body,
        out_shape=jax.ShapeDtypeStruct((NZ, 32, r // 32), jnp.int32),
        in_specs=[pl.BlockSpec(memory_space=pltpu.VMEM)],
        out_specs=pl.BlockSpec(memory_space=pltpu.VMEM),
        scratch_shapes=[pltpu.SemaphoreType.DMA, pltpu.SemaphoreType.DMA],
        compiler_params=pltpu.CompilerParams(collective_id=0),
    )(d2)
    return out.reshape(NZ, r)


def _a2av(x_send, meta):
    rows, d_model = x_send.shape

    xs3 = x_send.reshape(rows, 8, d_model // 8)

    def body(xs_ref, meta_ref, out_ref, sb_sem, ss_sem, rb_sem, rs_sem):
        my_x = lax.axis_index("x")
        my_y = lax.axis_index("y")
        my_z = lax.axis_index("z")

        barrier_sem = pltpu.get_barrier_semaphore()
        for dz in range(NZ):
            @pl.when(dz != my_z)
            def _():
                pl.semaphore_signal(
                    barrier_sem, inc=1,
                    device_id=(my_x, my_y, dz),
                    device_id_type=pl.DeviceIdType.MESH,
                )
        pl.semaphore_wait(barrier_sem, NZ - 1)

        for d in range(NZ):
            cnt = meta_ref[d]
            s0 = meta_ref[4 + d]
            t0 = meta_ref[8 + d]
            nfull = cnt // CHUNK
            rem = cnt - nfull * CHUNK

            @pl.when(d == my_z)
            def _():
                def cp_chunk(k, c):
                    out_ref[pl.ds(t0 + k * CHUNK, CHUNK), :, :] = (
                        xs_ref[pl.ds(s0 + k * CHUNK, CHUNK), :, :]
                    )
                    return c
                lax.fori_loop(0, nfull, cp_chunk, 0)

                def cp_row(k, c):
                    out_ref[pl.ds(t0 + nfull * CHUNK + k, 1), :, :] = (
                        xs_ref[pl.ds(s0 + nfull * CHUNK + k, 1), :, :]
                    )
                    return c
                lax.fori_loop(0, rem, cp_row, 0)

            @pl.when(d != my_z)
            def _():
                def send_chunk(k, c):
                    rdma = pltpu.make_async_remote_copy(
                        src_ref=xs_ref.at[pl.ds(s0 + k * CHUNK, CHUNK)],
                        dst_ref=out_ref.at[pl.ds(t0 + k * CHUNK, CHUNK)],
                        send_sem=sb_sem,
                        recv_sem=rb_sem,
                        device_id=(my_x, my_y, d),
                        device_id_type=pl.DeviceIdType.MESH,
                    )
                    rdma.start()
                    return c
                lax.fori_loop(0, nfull, send_chunk, 0)

                def send_row(k, c):
                    rdma = pltpu.make_async_remote_copy(
                        src_ref=xs_ref.at[pl.ds(s0 + nfull * CHUNK + k, 1)],
                        dst_ref=out_ref.at[pl.ds(t0 + nfull * CHUNK + k, 1)],
                        send_sem=ss_sem,
                        recv_sem=rs_sem,
                        device_id=(my_x, my_y, d),
                        device_id_type=pl.DeviceIdType.MESH,
                    )
                    rdma.start()
                    return c
                lax.fori_loop(0, rem, send_row, 0)

        dummy_big = pltpu.make_async_remote_copy(
            src_ref=xs_ref.at[pl.ds(0, CHUNK)],
            dst_ref=out_ref.at[pl.ds(0, CHUNK)],
            send_sem=sb_sem,
            recv_sem=rb_sem,
            device_id=(my_x, my_y, my_z),
            device_id_type=pl.DeviceIdType.MESH,
        )
        dummy_small = pltpu.make_async_remote_copy(
            src_ref=xs_ref.at[pl.ds(0, 1)],
            dst_ref=out_ref.at[pl.ds(0, 1)],
            send_sem=ss_sem,
            recv_sem=rs_sem,
            device_id=(my_x, my_y, my_z),
            device_id_type=pl.DeviceIdType.MESH,
        )

        def wait_sb(k, c):
            dummy_big.wait_send()
            return c
        lax.fori_loop(0, meta_ref[12], wait_sb, 0)

        def wait_ss(k, c):
            dummy_small.wait_send()
            return c
        lax.fori_loop(0, meta_ref[13], wait_ss, 0)

        def wait_rb(k, c):
            dummy_big.wait_recv()
            return c
        lax.fori_loop(0, meta_ref[14], wait_rb, 0)

        def wait_rs(k, c):
            dummy_small.wait_recv()
            return c
        lax.fori_loop(0, meta_ref[15], wait_rs, 0)

        @functools.partial(
            pl.run_scoped, second_barrier=pltpu.SemaphoreType.REGULAR
        )
        def _(second_barrier):
            for dz in range(NZ):
                @pl.when(dz != my_z)
                def _():
                    pl.semaphore_signal(
                        second_barrier, inc=1,
                        device_id=(my_x, my_y, dz),
                        device_id_type=pl.DeviceIdType.MESH,
                    )
            pl.semaphore_wait(second_barrier, NZ - 1)

    out = pl.pallas_call(
        body,
        out_shape=jax.ShapeDtypeStruct((rows, 8, d_model // 8), x_send.dtype),
        in_specs=[
            pl.BlockSpec(memory_space=pltpu.VMEM),
            pl.BlockSpec(memory_space=pltpu.SMEM),
        ],
        out_specs=pl.BlockSpec(memory_space=pltpu.VMEM),
        scratch_shapes=[
            pltpu.SemaphoreType.DMA,
            pltpu.SemaphoreType.DMA,
            pltpu.SemaphoreType.DMA,
            pltpu.SemaphoreType.DMA,
        ],
        compiler_params=pltpu.CompilerParams(collective_id=1),
    )(xs3, meta)
    return out.reshape(rows, d_model)


def kernel(x, dest):
    rows = x.shape[0]
    my_z = lax.axis_index("z")

    dest_full = _gather_dest(dest)

    c = (
        dest_full[:, :, None] == jnp.arange(NZ, dtype=dest_full.dtype)[None, None, :]
    ).sum(axis=1).astype(jnp.int32)

    cnt = c[my_z]
    csum = jnp.cumsum(cnt)
    src0 = jnp.concatenate([jnp.zeros((1,), jnp.int32), csum[:-1].astype(jnp.int32)])
    excl = (jnp.cumsum(c, axis=0) - c).astype(jnp.int32)
    dst0 = excl[my_z]

    recv_cnt = c[:, my_z]
    others = jnp.arange(NZ) != my_z
    zero = jnp.zeros((), jnp.int32)
    n_send_big = jnp.where(others, cnt // CHUNK, zero).sum()
    n_send_small = jnp.where(others, cnt % CHUNK, zero).sum()
    n_recv_big = jnp.where(others, recv_cnt // CHUNK, zero).sum()
    n_recv_small = jnp.where(others, recv_cnt % CHUNK, zero).sum()

    meta = jnp.concatenate(
        [
            cnt.astype(jnp.int32),
            src0,
            dst0,
            jnp.stack([n_send_big, n_send_small, n_recv_big, n_recv_small]),
        ]
    ).astype(jnp.int32)

    send_perm = jnp.argsort(dest, stable=True)
    x_send = jnp.take(x, send_perm, axis=0)

    return _a2av(x_send, meta)


# device time: 252248 ns/iter; 1.1158x vs baseline; 1.1158x over previous
import functools

import jax
import jax.numpy as jnp
from jax import lax
from jax.experimental import pallas as pl
from jax.experimental.pallas import tpu as pltpu

NZ = 4
CHUNK = 64


def _gather_dest(dest):
    r = dest.shape[0]
    d2 = dest.reshape(32, r // 32)

    def body(d_ref, out_ref, send_sem, recv_sem):
        my_x = lax.axis_index("x")
        my_y = lax.axis_index("y")
        my_z = lax.axis_index("z")

        barrier_sem = pltpu.get_barrier_semaphore()
        for dz in range(NZ):
            @pl.when(dz != my_z)
            def _():
                pl.semaphore_signal(
                    barrier_sem, inc=1,
                    device_id=(my_x, my_y, dz),
                    device_id_type=pl.DeviceIdType.MESH,
                )
        pl.semaphore_wait(barrier_sem, NZ - 1)

        out_ref[my_z, :, :] = d_ref[:, :]
        for dz in range(NZ):
            @pl.when(dz != my_z)
            def _():
                rdma = pltpu.make_async_remote_copy(
                    src_ref=d_ref,
                    dst_ref=out_ref.at[my_z],
                    send_sem=send_sem,
                    recv_sem=recv_sem,
                    device_id=(my_x, my_y, dz),
                    device_id_type=pl.DeviceIdType.MESH,
                )
                rdma.start()

        dummy = pltpu.make_async_remote_copy(
            src_ref=d_ref,
            dst_ref=out_ref.at[0],
            send_sem=send_sem,
            recv_sem=recv_sem,
            device_id=(my_x, my_y, my_z),
            device_id_type=pl.DeviceIdType.MESH,
        )
        for _ in range(NZ - 1):
            dummy.wait_send()
        for _ in range(NZ - 1):
            dummy.wait_recv()

    out = pl.pallas_call(
        body,
        out_shape=jax.ShapeDtypeStruct((NZ, 32, r // 32), jnp.int32),
        in_specs=[pl.BlockSpec(memory_space=pltpu.VMEM)],
        out_specs=pl.BlockSpec(memory_space=pltpu.VMEM),
        scratch_shapes=[pltpu.SemaphoreType.DMA, pltpu.SemaphoreType.DMA],
        compiler_params=pltpu.CompilerParams(collective_id=0),
    )(d2)
    return out.reshape(NZ, r)


def _a2av(x, perm, meta):
    rows, d_model = x.shape

    x3 = x.reshape(rows, 8, d_model // 8)

    def body(x_ref, perm_ref, meta_ref, out_ref, xs_ref,
             sb_sem, ss_sem, rb_sem, rs_sem):
        my_x = lax.axis_index("x")
        my_y = lax.axis_index("y")
        my_z = lax.axis_index("z")

        barrier_sem = pltpu.get_barrier_semaphore()
        for dz in range(NZ):
            @pl.when(dz != my_z)
            def _():
                pl.semaphore_signal(
                    barrier_sem, inc=1,
                    device_id=(my_x, my_y, dz),
                    device_id_type=pl.DeviceIdType.MESH,
                )
        pl.semaphore_wait(barrier_sem, NZ - 1)

        for d in range(NZ):
            cnt = meta_ref[d]
            s0 = meta_ref[4 + d]
            t0 = meta_ref[8 + d]
            nfull = cnt // CHUNK
            rem = cnt - nfull * CHUNK

            @pl.when(d == my_z)
            def _():
                def cp_row(k, c):
                    out_ref[pl.ds(t0 + k, 1), :, :] = (
                        x_ref[pl.ds(perm_ref[s0 + k], 1), :, :]
                    )
                    return c
                lax.fori_loop(0, cnt, cp_row, 0)

            @pl.when(d != my_z)
            def _():
                def gather_row(k, c):
                    xs_ref[pl.ds(s0 + k, 1), :, :] = (
                        x_ref[pl.ds(perm_ref[s0 + k], 1), :, :]
                    )
                    return c
                lax.fori_loop(0, cnt, gather_row, 0)

                def send_chunk(k, c):
                    rdma = pltpu.make_async_remote_copy(
                        src_ref=xs_ref.at[pl.ds(s0 + k * CHUNK, CHUNK)],
                        dst_ref=out_ref.at[pl.ds(t0 + k * CHUNK, CHUNK)],
                        send_sem=sb_sem,
                        recv_sem=rb_sem,
                        device_id=(my_x, my_y, d),
                        device_id_type=pl.DeviceIdType.MESH,
                    )
                    rdma.start()
                    return c
                lax.fori_loop(0, nfull, send_chunk, 0)

                def send_row(k, c):
                    rdma = pltpu.make_async_remote_copy(
                        src_ref=xs_ref.at[pl.ds(s0 + nfull * CHUNK + k, 1)],
                        dst_ref=out_ref.at[pl.ds(t0 + nfull * CHUNK + k, 1)],
                        send_sem=ss_sem,
                        recv_sem=rs_sem,
                        device_id=(my_x, my_y, d),
                        device_id_type=pl.DeviceIdType.MESH,
                    )
                    rdma.start()
                    return c
                lax.fori_loop(0, rem, send_row, 0)

        dummy_big = pltpu.make_async_remote_copy(
            src_ref=xs_ref.at[pl.ds(0, CHUNK)],
            dst_ref=out_ref.at[pl.ds(0, CHUNK)],
            send_sem=sb_sem,
            recv_sem=rb_sem,
            device_id=(my_x, my_y, my_z),
            device_id_type=pl.DeviceIdType.MESH,
        )
        dummy_small = pltpu.make_async_remote_copy(
            src_ref=xs_ref.at[pl.ds(0, 1)],
            dst_ref=out_ref.at[pl.ds(0, 1)],
            send_sem=ss_sem,
            recv_sem=rs_sem,
            device_id=(my_x, my_y, my_z),
            device_id_type=pl.DeviceIdType.MESH,
        )

        def wait_sb(k, c):
            dummy_big.wait_send()
            return c
        lax.fori_loop(0, meta_ref[12], wait_sb, 0)

        def wait_ss(k, c):
            dummy_small.wait_send()
            return c
        lax.fori_loop(0, meta_ref[13], wait_ss, 0)

        def wait_rb(k, c):
            dummy_big.wait_recv()
            return c
        lax.fori_loop(0, meta_ref[14], wait_rb, 0)

        def wait_rs(k, c):
            dummy_small.wait_recv()
            return c
        lax.fori_loop(0, meta_ref[15], wait_rs, 0)

        @functools.partial(
            pl.run_scoped, second_barrier=pltpu.SemaphoreType.REGULAR
        )
        def _(second_barrier):
            for dz in range(NZ):
                @pl.when(dz != my_z)
                def _():
                    pl.semaphore_signal(
                        second_barrier, inc=1,
                        device_id=(my_x, my_y, dz),
                        device_id_type=pl.DeviceIdType.MESH,
                    )
            pl.semaphore_wait(second_barrier, NZ - 1)

    out = pl.pallas_call(
        body,
        out_shape=jax.ShapeDtypeStruct((rows, 8, d_model // 8), x.dtype),
        in_specs=[
            pl.BlockSpec(memory_space=pltpu.VMEM),
            pl.BlockSpec(memory_space=pltpu.SMEM),
            pl.BlockSpec(memory_space=pltpu.SMEM),
        ],
        out_specs=pl.BlockSpec(memory_space=pltpu.VMEM),
        scratch_shapes=[
            pltpu.VMEM((rows, 8, d_model // 8), x.dtype),
            pltpu.SemaphoreType.DMA,
            pltpu.SemaphoreType.DMA,
            pltpu.SemaphoreType.DMA,
            pltpu.SemaphoreType.DMA,
        ],
        compiler_params=pltpu.CompilerParams(collective_id=1),
    )(x3, perm, meta)
    return out.reshape(rows, d_model)


def kernel(x, dest):
    rows = x.shape[0]
    my_z = lax.axis_index("z")

    dest_full = _gather_dest(dest)

    c = (
        dest_full[:, :, None] == jnp.arange(NZ, dtype=dest_full.dtype)[None, None, :]
    ).sum(axis=1).astype(jnp.int32)

    cnt = c[my_z]
    csum = jnp.cumsum(cnt)
    src0 = jnp.concatenate([jnp.zeros((1,), jnp.int32), csum[:-1].astype(jnp.int32)])
    excl = (jnp.cumsum(c, axis=0) - c).astype(jnp.int32)
    dst0 = excl[my_z]

    recv_cnt = c[:, my_z]
    others = jnp.arange(NZ) != my_z
    zero = jnp.zeros((), jnp.int32)
    n_send_big = jnp.where(others, cnt // CHUNK, zero).sum()
    n_send_small = jnp.where(others, cnt % CHUNK, zero).sum()
    n_recv_big = jnp.where(others, recv_cnt // CHUNK, zero).sum()
    n_recv_small = jnp.where(others, recv_cnt % CHUNK, zero).sum()

    meta = jnp.concatenate(
        [
            cnt.astype(jnp.int32),
            src0,
            dst0,
            jnp.stack([n_send_big, n_send_small, n_recv_big, n_recv_small]),
        ]
    ).astype(jnp.int32)

    send_perm = jnp.argsort(dest, stable=True).astype(jnp.int32)

    return _a2av(x, send_perm, meta)
